# baseline (speedup 1.0000x reference)
import jax
import jax.numpy as jnp
from jax.experimental import pallas as pl
from jax.experimental.pallas import tpu as pltpu


def _add_body(ids_ref, hid_ref, emb_ref, gate_ref, out_ref):
    t = pl.program_id(1)
    g = jnp.tanh(gate_ref[0, 0])
    row = emb_ref[0, t, :]
    out_ref[...] = hid_ref[...] + g * row.reshape(1, 1, 1, emb_ref.shape[-1])


def kernel(hidden_state, aspect_ratio_ids, embedding_table, gate):
    B, T, P, H = hidden_state.shape
    V = embedding_table.shape[0]
    table = embedding_table.reshape(V, T, H)
    gate2d = gate.reshape(1, 1)
    ids = aspect_ratio_ids.astype(jnp.int32)

    grid_spec = pltpu.PrefetchScalarGridSpec(
        num_scalar_prefetch=1,
        grid=(B, T),
        in_specs=[
            pl.BlockSpec((1, 1, P, H), lambda b, t, ids: (b, t, 0, 0)),
            pl.BlockSpec((1, T, H), lambda b, t, ids: (ids[b], 0, 0)),
            pl.BlockSpec(memory_space=pltpu.SMEM),
        ],
        out_specs=pl.BlockSpec((1, 1, P, H), lambda b, t, ids: (b, t, 0, 0)),
    )
    out = pl.pallas_call(
        _add_body,
        grid_spec=grid_spec,
        out_shape=jax.ShapeDtypeStruct((B, T, P, H), hidden_state.dtype),
    )(ids, hidden_state, table, gate2d)
    return out
